# Initial kernel scaffold; baseline (speedup 1.0000x reference)
import jax
import jax.numpy as jnp
from jax.experimental import pallas as pl


def kernel(x, W_router, W_gate, W_up, W_down):
    raise NotImplementedError("write your pallas kernel here")


# speedup vs baseline: 2.4553x; 2.4553x over previous
import functools

import jax
import jax.numpy as jnp
from jax import lax
from jax.experimental import pallas as pl
from jax.experimental.pallas import tpu as pltpu
from jax.experimental.pallas import tpu_sc as plsc

T = 2048
D = 768
F = 2048
E = 16
K = 2
P = T * K
TILE = 256
MAX_TILES = 32
MAX_ROWS = MAX_TILES * TILE

NC, NS = 2, 16
NW = NC * NS
PPW = P // NW
TPW = T // NW
CH = 32


def _route_body(x_ref, wr_ref, pos_ref, dw_ref, texp_ref, nt_ref,
                oh_ref):
    x = x_ref[...]
    wr = wr_ref[...]
    logits = lax.dot_general(x, wr, (((1,), (1,)), ((), ())),
                             preferred_element_type=jnp.float32)

    col = lax.broadcasted_iota(jnp.int32, (T, E), 1)
    m1 = jnp.max(logits, axis=1, keepdims=True)
    a1 = jnp.min(jnp.where(logits == m1, col, E), axis=1, keepdims=True)
    masked = jnp.where(col == a1, -jnp.inf, logits)
    m2 = jnp.max(masked, axis=1, keepdims=True)
    a2 = jnp.min(jnp.where(masked == m2, col, E), axis=1, keepdims=True)

    d0 = 1.0 / (1.0 + jnp.exp(m2 - m1))
    dw_ref[:T, :] = jnp.broadcast_to(d0, (T, 128))
    dw_ref[T:, :] = jnp.broadcast_to(1.0 - d0, (T, 128))

    oh1 = (col == a1).astype(jnp.float32)
    oh2 = (col == a2).astype(jnp.float32)
    oh_ref[:T, :] = oh1
    oh_ref[T:, :] = oh2

    counts = (jnp.sum(oh1, axis=0, keepdims=True)
              + jnp.sum(oh2, axis=0, keepdims=True))
    tpe = jnp.floor((counts + (TILE - 1)) * (1.0 / TILE))
    r16 = lax.broadcasted_iota(jnp.int32, (E, E), 0)
    c16 = lax.broadcasted_iota(jnp.int32, (E, E), 1)
    ustrict = (r16 < c16).astype(jnp.float32)
    tile_base = lax.dot_general(tpe, ustrict, (((1,), (0,)), ((), ())),
                                preferred_element_type=jnp.float32)
    row_base = tile_base * TILE
    nt = jnp.sum(tpe)
    nt_ref[...] = jnp.broadcast_to(nt.astype(jnp.int32), (1, 1))

    trow = lax.broadcasted_iota(jnp.int32, (MAX_TILES, E), 0)
    t_eff = jnp.minimum(trow, nt.astype(jnp.int32) - 1)
    ge = (t_eff >= tile_base.astype(jnp.int32)).astype(jnp.int32)
    texp_ref[...] = jnp.sum(ge, axis=1, keepdims=True) - 1

    rr = lax.broadcasted_iota(jnp.int32, (TILE, TILE), 0)
    cc = lax.broadcasted_iota(jnp.int32, (TILE, TILE), 1)
    tri = (cc <= rr).astype(jnp.float32)

    def blk(b, base):
        ohb = oh_ref[pl.ds(b * TILE, TILE), :]
        csum = lax.dot_general(tri, ohb, (((1,), (0,)), ((), ())),
                               preferred_element_type=jnp.float32)
        rank = csum - 1.0 + base
        posb = jnp.sum((rank + row_base) * ohb, axis=1, keepdims=True)
        pos_ref[pl.ds(b * TILE, TILE), :] = posb.astype(jnp.int32)
        return base + csum[TILE - 1:TILE, :]

    lax.fori_loop(0, P // TILE, blk, jnp.zeros((1, E), jnp.float32))


def _route(x_flat, w_router):
    return pl.pallas_call(
        _route_body,
        out_shape=[
            jax.ShapeDtypeStruct((P, 1), jnp.int32),
            jax.ShapeDtypeStruct((P, 128), jnp.float32),
            jax.ShapeDtypeStruct((MAX_TILES, 1), jnp.int32),
            jax.ShapeDtypeStruct((1, 1), jnp.int32),
        ],
        scratch_shapes=[pltpu.VMEM((P, E), jnp.float32)],
    )(x_flat, w_router)


def _dispatch_body(x_hbm, pos_hbm, dw_hbm, xs_hbm, ws_hbm, pos_v, rows_v,
                   dw_v):
    wid = lax.axis_index("s") * NC + lax.axis_index("c")
    base = wid * PPW
    tok = lax.rem(base, T)
    pltpu.sync_copy(pos_hbm.at[pl.ds(base, PPW)], pos_v)
    pltpu.sync_copy(dw_hbm.at[pl.ds(base, PPW)], dw_v)
    pltpu.sync_copy(x_hbm.at[pl.ds(tok, PPW)], rows_v)
    pltpu.sync_copy(rows_v, xs_hbm.at[pos_v])
    pltpu.sync_copy(dw_v, ws_hbm.at[pos_v])


@functools.cache
def _dispatch():
    return pl.kernel(
        _dispatch_body,
        out_type=[
            jax.ShapeDtypeStruct((MAX_ROWS, D), jnp.float32),
            jax.ShapeDtypeStruct((MAX_ROWS, 128), jnp.float32),
        ],
        mesh=plsc.VectorSubcoreMesh(core_axis_name="c", subcore_axis_name="s",
                                    num_cores=NC, num_subcores=NS),
        scratch_types=[
            pltpu.VMEM((PPW,), jnp.int32),
            pltpu.VMEM((PPW, D), jnp.float32),
            pltpu.VMEM((PPW, 128), jnp.float32),
        ],
    )


def _ffn_body(texp_s, nt_s, x_ref, ws_ref, wg_ref, wu_ref, wd_ref, o_ref):
    del texp_s
    t = pl.program_id(0)

    @pl.when(t < nt_s[0])
    def _():
        x = x_ref[...]
        g = lax.dot_general(x, wg_ref[0], (((1,), (1,)), ((), ())),
                            preferred_element_type=jnp.float32)
        u = lax.dot_general(x, wu_ref[0], (((1,), (1,)), ((), ())),
                            preferred_element_type=jnp.float32)
        h = g * (1.0 / (1.0 + jnp.exp(-g))) * u
        o = lax.dot_general(h, wd_ref[0], (((1,), (1,)), ((), ())),
                            preferred_element_type=jnp.float32)
        o_ref[...] = o * ws_ref[:, 0:1]


def _ffn(texp, nt, xs, ws, w_gate, w_up, w_down):
    grid_spec = pltpu.PrefetchScalarGridSpec(
        num_scalar_prefetch=2,
        grid=(MAX_TILES,),
        in_specs=[
            pl.BlockSpec((TILE, D),
                         lambda t, texp, nt: (jnp.minimum(t, nt[0] - 1), 0)),
            pl.BlockSpec((TILE, 128),
                         lambda t, texp, nt: (jnp.minimum(t, nt[0] - 1), 0)),
            pl.BlockSpec((1, F, D), lambda t, texp, nt: (texp[t], 0, 0)),
            pl.BlockSpec((1, F, D), lambda t, texp, nt: (texp[t], 0, 0)),
            pl.BlockSpec((1, D, F), lambda t, texp, nt: (texp[t], 0, 0)),
        ],
        out_specs=pl.BlockSpec(
            (TILE, D), lambda t, texp, nt: (jnp.minimum(t, nt[0] - 1), 0)),
    )
    return pl.pallas_call(
        _ffn_body,
        grid_spec=grid_spec,
        out_shape=jax.ShapeDtypeStruct((MAX_ROWS, D), jnp.float32),
    )(texp, nt, xs, ws, w_gate, w_up, w_down)


def _combine_body(os_hbm, pos_hbm, out_hbm,
                  pos0_v, pos1_v, ra_v, rb_v, out_v, sem):
    wid = lax.axis_index("s") * NC + lax.axis_index("c")
    for c in range(TPW // CH):
        n0 = wid * TPW + c * CH
        pltpu.sync_copy(pos_hbm.at[pl.ds(n0, CH)], pos0_v)
        pltpu.sync_copy(pos_hbm.at[pl.ds(T + n0, CH)], pos1_v)
        pltpu.async_copy(os_hbm.at[pos0_v], ra_v, sem).wait()
        pltpu.async_copy(os_hbm.at[pos1_v], rb_v, sem).wait()

        def tokstep(j, carry):
            for q in range(D // 16):
                a = ra_v[j, pl.ds(q * 16, 16)]
                b = rb_v[j, pl.ds(q * 16, 16)]
                out_v[j, pl.ds(q * 16, 16)] = a + b
            return carry

        lax.fori_loop(0, CH, tokstep, 0)
        pltpu.sync_copy(out_v, out_hbm.at[pl.ds(n0, CH)])


@functools.cache
def _combine():
    return pl.kernel(
        _combine_body,
        out_type=jax.ShapeDtypeStruct((T, D), jnp.float32),
        mesh=plsc.VectorSubcoreMesh(core_axis_name="c", subcore_axis_name="s",
                                    num_cores=NC, num_subcores=NS),
        scratch_types=[
            pltpu.VMEM((CH,), jnp.int32),
            pltpu.VMEM((CH,), jnp.int32),
            pltpu.VMEM((CH, D), jnp.float32),
            pltpu.VMEM((CH, D), jnp.float32),
            pltpu.VMEM((CH, D), jnp.float32),
            pltpu.SemaphoreType.DMA,
        ],
    )


def kernel(x, W_router, W_gate, W_up, W_down):
    x_flat = x.reshape(T, D)
    pos, dw, texp, nt = _route(x_flat, W_router)
    pos_f = pos.reshape(P)
    xs, ws = _dispatch()(x_flat, pos_f, dw)
    os_ = _ffn(texp.reshape(MAX_TILES), nt.reshape(1), xs, ws,
               W_gate, W_up, W_down)
    out = _combine()(os_, pos_f)
    return out.reshape(x.shape)


# speedup vs baseline: 2.4626x; 1.0030x over previous
import functools

import jax
import jax.numpy as jnp
from jax import lax
from jax.experimental import pallas as pl
from jax.experimental.pallas import tpu as pltpu
from jax.experimental.pallas import tpu_sc as plsc

T = 2048
D = 768
F = 2048
E = 16
K = 2
P = T * K
TILE = 256
MAX_TILES = 32
MAX_ROWS = MAX_TILES * TILE

NC, NS = 2, 16
NW = NC * NS
PPW = P // NW
TPW = T // NW
CH = 32


def _route_body(x_ref, wr_ref, pos_ref, dw_ref, texp_ref, nt_ref,
                oh_ref):
    x = x_ref[...]
    wr = wr_ref[...]
    logits = lax.dot_general(x, wr, (((1,), (1,)), ((), ())),
                             preferred_element_type=jnp.float32)

    col = lax.broadcasted_iota(jnp.int32, (T, E), 1)
    m1 = jnp.max(logits, axis=1, keepdims=True)
    a1 = jnp.min(jnp.where(logits == m1, col, E), axis=1, keepdims=True)
    masked = jnp.where(col == a1, -jnp.inf, logits)
    m2 = jnp.max(masked, axis=1, keepdims=True)
    a2 = jnp.min(jnp.where(masked == m2, col, E), axis=1, keepdims=True)

    d0 = 1.0 / (1.0 + jnp.exp(m2 - m1))
    dw_ref[:T, :] = jnp.broadcast_to(d0, (T, 128))
    dw_ref[T:, :] = jnp.broadcast_to(1.0 - d0, (T, 128))

    oh1 = (col == a1).astype(jnp.float32)
    oh2 = (col == a2).astype(jnp.float32)
    oh_ref[:T, :] = oh1
    oh_ref[T:, :] = oh2

    counts = (jnp.sum(oh1, axis=0, keepdims=True)
              + jnp.sum(oh2, axis=0, keepdims=True))
    tpe = jnp.floor((counts + (TILE - 1)) * (1.0 / TILE))
    r16 = lax.broadcasted_iota(jnp.int32, (E, E), 0)
    c16 = lax.broadcasted_iota(jnp.int32, (E, E), 1)
    ustrict = (r16 < c16).astype(jnp.float32)
    tile_base = lax.dot_general(tpe, ustrict, (((1,), (0,)), ((), ())),
                                preferred_element_type=jnp.float32)
    row_base = tile_base * TILE
    nt = jnp.sum(tpe)
    nt_ref[...] = jnp.broadcast_to(nt.astype(jnp.int32), (1, 1))

    trow = lax.broadcasted_iota(jnp.int32, (MAX_TILES, E), 0)
    t_eff = jnp.minimum(trow, nt.astype(jnp.int32) - 1)
    ge = (t_eff >= tile_base.astype(jnp.int32)).astype(jnp.int32)
    texp_ref[...] = jnp.sum(ge, axis=1, keepdims=True) - 1

    rr = lax.broadcasted_iota(jnp.int32, (TILE, TILE), 0)
    cc = lax.broadcasted_iota(jnp.int32, (TILE, TILE), 1)
    tri = (cc <= rr).astype(jnp.float32)

    def blk(b, base):
        ohb = oh_ref[pl.ds(b * TILE, TILE), :]
        csum = lax.dot_general(tri, ohb, (((1,), (0,)), ((), ())),
                               preferred_element_type=jnp.float32)
        rank = csum - 1.0 + base
        posb = jnp.sum((rank + row_base) * ohb, axis=1, keepdims=True)
        pos_ref[pl.ds(b * TILE, TILE), :] = posb.astype(jnp.int32)
        return base + csum[TILE - 1:TILE, :]

    lax.fori_loop(0, P // TILE, blk, jnp.zeros((1, E), jnp.float32))


def _route(x_flat, w_router):
    return pl.pallas_call(
        _route_body,
        out_shape=[
            jax.ShapeDtypeStruct((P, 1), jnp.int32),
            jax.ShapeDtypeStruct((P, 128), jnp.float32),
            jax.ShapeDtypeStruct((MAX_TILES, 1), jnp.int32),
            jax.ShapeDtypeStruct((1, 1), jnp.int32),
        ],
        scratch_shapes=[pltpu.VMEM((P, E), jnp.float32)],
    )(x_flat, w_router)


def _dispatch_body(x_hbm, pos_hbm, dw_hbm, xs_hbm, ws_hbm, pos_v, rows_v,
                   dw_v):
    wid = lax.axis_index("s") * NC + lax.axis_index("c")
    base = wid * PPW
    tok = lax.rem(base, T)
    pltpu.sync_copy(pos_hbm.at[pl.ds(base, PPW)], pos_v)
    pltpu.sync_copy(dw_hbm.at[pl.ds(base, PPW)], dw_v)
    pltpu.sync_copy(x_hbm.at[pl.ds(tok, PPW)], rows_v)
    pltpu.sync_copy(rows_v, xs_hbm.at[pos_v])
    pltpu.sync_copy(dw_v, ws_hbm.at[pos_v])


@functools.cache
def _dispatch():
    return pl.kernel(
        _dispatch_body,
        out_type=[
            jax.ShapeDtypeStruct((MAX_ROWS, D), jnp.float32),
            jax.ShapeDtypeStruct((MAX_ROWS, 128), jnp.float32),
        ],
        mesh=plsc.VectorSubcoreMesh(core_axis_name="c", subcore_axis_name="s",
                                    num_cores=NC, num_subcores=NS),
        scratch_types=[
            pltpu.VMEM((PPW,), jnp.int32),
            pltpu.VMEM((PPW, D), jnp.float32),
            pltpu.VMEM((PPW, 128), jnp.float32),
        ],
    )


def _ffn_body(texp_s, nt_s, x_ref, ws_ref, wg_ref, wu_ref, wd_ref, o_ref):
    del texp_s
    t = pl.program_id(0)

    @pl.when(t < nt_s[0])
    def _():
        x = x_ref[...].astype(jnp.bfloat16)
        g = lax.dot_general(x, wg_ref[0].astype(jnp.bfloat16),
                            (((1,), (1,)), ((), ())),
                            preferred_element_type=jnp.float32)
        u = lax.dot_general(x, wu_ref[0].astype(jnp.bfloat16),
                            (((1,), (1,)), ((), ())),
                            preferred_element_type=jnp.float32)
        h = g * (1.0 / (1.0 + jnp.exp(-g))) * u
        o = lax.dot_general(h.astype(jnp.bfloat16),
                            wd_ref[0].astype(jnp.bfloat16),
                            (((1,), (1,)), ((), ())),
                            preferred_element_type=jnp.float32)
        o_ref[...] = o * ws_ref[:, 0:1]


def _ffn(texp, nt, xs, ws, w_gate, w_up, w_down):
    grid_spec = pltpu.PrefetchScalarGridSpec(
        num_scalar_prefetch=2,
        grid=(MAX_TILES,),
        in_specs=[
            pl.BlockSpec((TILE, D),
                         lambda t, texp, nt: (jnp.minimum(t, nt[0] - 1), 0)),
            pl.BlockSpec((TILE, 128),
                         lambda t, texp, nt: (jnp.minimum(t, nt[0] - 1), 0)),
            pl.BlockSpec((1, F, D), lambda t, texp, nt: (texp[t], 0, 0)),
            pl.BlockSpec((1, F, D), lambda t, texp, nt: (texp[t], 0, 0)),
            pl.BlockSpec((1, D, F), lambda t, texp, nt: (texp[t], 0, 0)),
        ],
        out_specs=pl.BlockSpec(
            (TILE, D), lambda t, texp, nt: (jnp.minimum(t, nt[0] - 1), 0)),
    )
    return pl.pallas_call(
        _ffn_body,
        grid_spec=grid_spec,
        out_shape=jax.ShapeDtypeStruct((MAX_ROWS, D), jnp.float32),
    )(texp, nt, xs, ws, w_gate, w_up, w_down)


def _combine_body(os_hbm, pos_hbm, out_hbm,
                  pos0_v, pos1_v, ra_v, rb_v, out_v, sem):
    wid = lax.axis_index("s") * NC + lax.axis_index("c")
    for c in range(TPW // CH):
        n0 = wid * TPW + c * CH
        pltpu.sync_copy(pos_hbm.at[pl.ds(n0, CH)], pos0_v)
        pltpu.sync_copy(pos_hbm.at[pl.ds(T + n0, CH)], pos1_v)
        pltpu.async_copy(os_hbm.at[pos0_v], ra_v, sem).wait()
        pltpu.async_copy(os_hbm.at[pos1_v], rb_v, sem).wait()

        def tokstep(j, carry):
            for q in range(D // 16):
                a = ra_v[j, pl.ds(q * 16, 16)]
                b = rb_v[j, pl.ds(q * 16, 16)]
                out_v[j, pl.ds(q * 16, 16)] = a + b
            return carry

        lax.fori_loop(0, CH, tokstep, 0)
        pltpu.sync_copy(out_v, out_hbm.at[pl.ds(n0, CH)])


@functools.cache
def _combine():
    return pl.kernel(
        _combine_body,
        out_type=jax.ShapeDtypeStruct((T, D), jnp.float32),
        mesh=plsc.VectorSubcoreMesh(core_axis_name="c", subcore_axis_name="s",
                                    num_cores=NC, num_subcores=NS),
        scratch_types=[
            pltpu.VMEM((CH,), jnp.int32),
            pltpu.VMEM((CH,), jnp.int32),
            pltpu.VMEM((CH, D), jnp.float32),
            pltpu.VMEM((CH, D), jnp.float32),
            pltpu.VMEM((CH, D), jnp.float32),
            pltpu.SemaphoreType.DMA,
        ],
    )


def kernel(x, W_router, W_gate, W_up, W_down):
    x_flat = x.reshape(T, D)
    pos, dw, texp, nt = _route(x_flat, W_router)
    pos_f = pos.reshape(P)
    xs, ws = _dispatch()(x_flat, pos_f, dw)
    os_ = _ffn(texp.reshape(MAX_TILES), nt.reshape(1), xs, ws,
               W_gate, W_up, W_down)
    out = _combine()(os_, pos_f)
    return out.reshape(x.shape)


# speedup vs baseline: 3.1072x; 1.2618x over previous
import functools

import jax
import jax.numpy as jnp
from jax import lax
from jax.experimental import pallas as pl
from jax.experimental.pallas import tpu as pltpu
from jax.experimental.pallas import tpu_sc as plsc

T = 2048
D = 768
F = 2048
E = 16
K = 2
P = T * K
TILE = 384
MAX_TILES = 26
RB = 1024
MAX_ROWS = MAX_TILES * TILE

NC, NS = 2, 16
NW = NC * NS
PPW = P // NW
TPW = T // NW
CH = 32


def _route_body(x_ref, wr_ref, pos_ref, dw_ref, texp_ref, nt_ref,
                oh_ref):
    x = x_ref[...]
    wr = wr_ref[...]
    logits = lax.dot_general(x, wr, (((1,), (1,)), ((), ())),
                             preferred_element_type=jnp.float32)

    col = lax.broadcasted_iota(jnp.int32, (T, E), 1)
    m1 = jnp.max(logits, axis=1, keepdims=True)
    a1 = jnp.min(jnp.where(logits == m1, col, E), axis=1, keepdims=True)
    masked = jnp.where(col == a1, -jnp.inf, logits)
    m2 = jnp.max(masked, axis=1, keepdims=True)
    a2 = jnp.min(jnp.where(masked == m2, col, E), axis=1, keepdims=True)

    d0 = 1.0 / (1.0 + jnp.exp(m2 - m1))
    dw_ref[:T, :] = jnp.broadcast_to(d0, (T, 128))
    dw_ref[T:, :] = jnp.broadcast_to(1.0 - d0, (T, 128))

    oh1 = (col == a1).astype(jnp.float32)
    oh2 = (col == a2).astype(jnp.float32)
    oh_ref[:T, :] = oh1
    oh_ref[T:, :] = oh2

    counts = (jnp.sum(oh1, axis=0, keepdims=True)
              + jnp.sum(oh2, axis=0, keepdims=True))
    tpe = jnp.floor((counts + (TILE - 1)) * (1.0 / TILE))
    r16 = lax.broadcasted_iota(jnp.int32, (E, E), 0)
    c16 = lax.broadcasted_iota(jnp.int32, (E, E), 1)
    ustrict = (r16 < c16).astype(jnp.float32)
    tile_base = lax.dot_general(tpe, ustrict, (((1,), (0,)), ((), ())),
                                preferred_element_type=jnp.float32)
    row_base = tile_base * TILE
    nt = jnp.sum(tpe)
    nt_ref[...] = jnp.broadcast_to(nt.astype(jnp.int32), (1, 1))

    trow = lax.broadcasted_iota(jnp.int32, (MAX_TILES, E), 0)
    t_eff = jnp.minimum(trow, nt.astype(jnp.int32) - 1)
    ge = (t_eff >= tile_base.astype(jnp.int32)).astype(jnp.int32)
    texp_ref[...] = jnp.sum(ge, axis=1, keepdims=True) - 1

    rr = lax.broadcasted_iota(jnp.int32, (RB, RB), 0)
    cc = lax.broadcasted_iota(jnp.int32, (RB, RB), 1)
    tri = (cc <= rr).astype(jnp.float32)

    def blk(b, base):
        ohb = oh_ref[pl.ds(b * RB, RB), :]
        csum = lax.dot_general(tri, ohb, (((1,), (0,)), ((), ())),
                               preferred_element_type=jnp.float32)
        rank = csum - 1.0 + base
        posb = jnp.sum((rank + row_base) * ohb, axis=1, keepdims=True)
        pos_ref[pl.ds(b * RB, RB), :] = posb.astype(jnp.int32)
        return base + csum[RB - 1:RB, :]

    lax.fori_loop(0, P // RB, blk, jnp.zeros((1, E), jnp.float32))


def _route(x_flat, w_router):
    return pl.pallas_call(
        _route_body,
        out_shape=[
            jax.ShapeDtypeStruct((P, 1), jnp.int32),
            jax.ShapeDtypeStruct((P, 128), jnp.float32),
            jax.ShapeDtypeStruct((MAX_TILES, 1), jnp.int32),
            jax.ShapeDtypeStruct((1, 1), jnp.int32),
        ],
        scratch_shapes=[pltpu.VMEM((P, E), jnp.float32)],
    )(x_flat, w_router)


def _dispatch_body(x_hbm, pos_hbm, dw_hbm, xs_hbm, ws_hbm,
                   pos0_v, pos1_v, rows_v, dw0_v, dw1_v, sem_in, sem_out):
    wid = lax.axis_index("s") * NC + lax.axis_index("c")
    n0 = wid * TPW
    c1 = pltpu.async_copy(pos_hbm.at[pl.ds(n0, TPW)], pos0_v, sem_in)
    c2 = pltpu.async_copy(pos_hbm.at[pl.ds(T + n0, TPW)], pos1_v, sem_in)
    c3 = pltpu.async_copy(dw_hbm.at[pl.ds(n0, TPW)], dw0_v, sem_in)
    c4 = pltpu.async_copy(dw_hbm.at[pl.ds(T + n0, TPW)], dw1_v, sem_in)
    c5 = pltpu.async_copy(x_hbm.at[pl.ds(n0, TPW)], rows_v, sem_in)
    c1.wait(); c2.wait(); c3.wait(); c4.wait(); c5.wait()
    s1 = pltpu.async_copy(rows_v, xs_hbm.at[pos0_v], sem_out)
    s2 = pltpu.async_copy(rows_v, xs_hbm.at[pos1_v], sem_out)
    s3 = pltpu.async_copy(dw0_v, ws_hbm.at[pos0_v], sem_out)
    s4 = pltpu.async_copy(dw1_v, ws_hbm.at[pos1_v], sem_out)
    s1.wait(); s2.wait(); s3.wait(); s4.wait()


@functools.cache
def _dispatch():
    return pl.kernel(
        _dispatch_body,
        out_type=[
            jax.ShapeDtypeStruct((MAX_ROWS, D), jnp.float32),
            jax.ShapeDtypeStruct((MAX_ROWS, 128), jnp.float32),
        ],
        mesh=plsc.VectorSubcoreMesh(core_axis_name="c", subcore_axis_name="s",
                                    num_cores=NC, num_subcores=NS),
        scratch_types=[
            pltpu.VMEM((TPW,), jnp.int32),
            pltpu.VMEM((TPW,), jnp.int32),
            pltpu.VMEM((TPW, D), jnp.float32),
            pltpu.VMEM((TPW, 128), jnp.float32),
            pltpu.VMEM((TPW, 128), jnp.float32),
            pltpu.SemaphoreType.DMA,
            pltpu.SemaphoreType.DMA,
        ],
    )


def _ffn_body(texp_s, nt_s, x_ref, ws_ref, wg_ref, wu_ref, wd_ref, o_ref):
    del texp_s
    t = pl.program_id(0)

    @pl.when(t < nt_s[0, 0])
    def _():
        x = x_ref[...].astype(jnp.bfloat16)
        g = lax.dot_general(x, wg_ref[0].astype(jnp.bfloat16),
                            (((1,), (1,)), ((), ())),
                            preferred_element_type=jnp.float32)
        u = lax.dot_general(x, wu_ref[0].astype(jnp.bfloat16),
                            (((1,), (1,)), ((), ())),
                            preferred_element_type=jnp.float32)
        h = g * (1.0 / (1.0 + jnp.exp(-g))) * u
        o = lax.dot_general(h.astype(jnp.bfloat16),
                            wd_ref[0].astype(jnp.bfloat16),
                            (((1,), (1,)), ((), ())),
                            preferred_element_type=jnp.float32)
        o_ref[...] = o * ws_ref[:, 0:1]


def _ffn(texp, nt, xs, ws, w_gate, w_up, w_down):
    grid_spec = pltpu.PrefetchScalarGridSpec(
        num_scalar_prefetch=2,
        grid=(MAX_TILES,),
        in_specs=[
            pl.BlockSpec((TILE, D),
                         lambda t, texp, nt: (jnp.minimum(t, nt[0, 0] - 1), 0)),
            pl.BlockSpec((TILE, 128),
                         lambda t, texp, nt: (jnp.minimum(t, nt[0, 0] - 1), 0)),
            pl.BlockSpec((1, F, D), lambda t, texp, nt: (texp[t, 0], 0, 0)),
            pl.BlockSpec((1, F, D), lambda t, texp, nt: (texp[t, 0], 0, 0)),
            pl.BlockSpec((1, D, F), lambda t, texp, nt: (texp[t, 0], 0, 0)),
        ],
        out_specs=pl.BlockSpec(
            (TILE, D), lambda t, texp, nt: (jnp.minimum(t, nt[0, 0] - 1), 0)),
    )
    return pl.pallas_call(
        _ffn_body,
        grid_spec=grid_spec,
        out_shape=jax.ShapeDtypeStruct((MAX_ROWS, D), jnp.float32),
    )(texp, nt, xs, ws, w_gate, w_up, w_down)


def _combine_body(os_hbm, pos_hbm, out_hbm,
                  pos_vs, ra_vs, rb_vs, semg0, semg1, semw):
    wid = lax.axis_index("s") * NC + lax.axis_index("c")
    semg = (semg0, semg1)
    gathers = []
    for c in range(TPW // CH):
        n0 = wid * TPW + c * CH
        pltpu.sync_copy(pos_hbm.at[pl.ds(n0, CH)], pos_vs[2 * c])
        pltpu.sync_copy(pos_hbm.at[pl.ds(T + n0, CH)], pos_vs[2 * c + 1])
        ga = pltpu.async_copy(os_hbm.at[pos_vs[2 * c]], ra_vs[c], semg[c])
        gb = pltpu.async_copy(os_hbm.at[pos_vs[2 * c + 1]], rb_vs[c], semg[c])
        gathers.append((ga, gb))

    writes = []
    for c in range(TPW // CH):
        n0 = wid * TPW + c * CH
        ga, gb = gathers[c]
        ga.wait(); gb.wait()
        ra_v, rb_v = ra_vs[c], rb_vs[c]

        def tokstep(j, carry):
            for q in range(D // 16):
                a = ra_v[j, pl.ds(q * 16, 16)]
                b = rb_v[j, pl.ds(q * 16, 16)]
                ra_v[j, pl.ds(q * 16, 16)] = a + b
            return carry

        lax.fori_loop(0, CH, tokstep, 0)
        writes.append(pltpu.async_copy(ra_v, out_hbm.at[pl.ds(n0, CH)], semw))
    for w in writes:
        w.wait()


@functools.cache
def _combine():
    nch = TPW // CH
    return pl.kernel(
        _combine_body,
        out_type=jax.ShapeDtypeStruct((T, D), jnp.float32),
        mesh=plsc.VectorSubcoreMesh(core_axis_name="c", subcore_axis_name="s",
                                    num_cores=NC, num_subcores=NS),
        scratch_types=[
            [pltpu.VMEM((CH,), jnp.int32) for _ in range(2 * nch)],
            [pltpu.VMEM((CH, D), jnp.float32) for _ in range(nch)],
            [pltpu.VMEM((CH, D), jnp.float32) for _ in range(nch)],
            pltpu.SemaphoreType.DMA,
            pltpu.SemaphoreType.DMA,
            pltpu.SemaphoreType.DMA,
        ],
    )


def kernel(x, W_router, W_gate, W_up, W_down):
    x_flat = x.reshape(T, D)
    pos, dw, texp, nt = _route(x_flat, W_router)
    pos_f = pos.reshape(P)
    xs, ws = _dispatch()(x_flat, pos_f, dw)
    os_ = _ffn(texp, nt, xs, ws, W_gate, W_up, W_down)
    out = _combine()(os_, pos_f)
    return out.reshape(x.shape)


# speedup vs baseline: 3.1504x; 1.0139x over previous
import functools

import jax
import jax.numpy as jnp
from jax import lax
from jax.experimental import pallas as pl
from jax.experimental.pallas import tpu as pltpu
from jax.experimental.pallas import tpu_sc as plsc

T = 2048
D = 768
F = 2048
E = 16
K = 2
P = T * K
TILE = 320
MAX_TILES = 28
RB = 1024
MAX_ROWS = MAX_TILES * TILE

NC, NS = 2, 16
NW = NC * NS
PPW = P // NW
TPW = T // NW
CH = 32


def _route_body(x_ref, wr_ref, pos_ref, dw_ref, texp_ref, nt_ref,
                oh_ref):
    x = x_ref[...]
    wr = wr_ref[...]
    logits = lax.dot_general(x, wr, (((1,), (1,)), ((), ())),
                             preferred_element_type=jnp.float32)

    col = lax.broadcasted_iota(jnp.int32, (T, E), 1)
    m1 = jnp.max(logits, axis=1, keepdims=True)
    a1 = jnp.min(jnp.where(logits == m1, col, E), axis=1, keepdims=True)
    masked = jnp.where(col == a1, -jnp.inf, logits)
    m2 = jnp.max(masked, axis=1, keepdims=True)
    a2 = jnp.min(jnp.where(masked == m2, col, E), axis=1, keepdims=True)

    d0 = 1.0 / (1.0 + jnp.exp(m2 - m1))
    dw_ref[:T, :] = jnp.broadcast_to(d0, (T, 128))
    dw_ref[T:, :] = jnp.broadcast_to(1.0 - d0, (T, 128))

    oh1 = (col == a1).astype(jnp.float32)
    oh2 = (col == a2).astype(jnp.float32)
    oh_ref[:T, :] = oh1
    oh_ref[T:, :] = oh2

    counts = (jnp.sum(oh1, axis=0, keepdims=True)
              + jnp.sum(oh2, axis=0, keepdims=True))
    tpe = jnp.floor((counts + (TILE - 1)) * (1.0 / TILE))
    r16 = lax.broadcasted_iota(jnp.int32, (E, E), 0)
    c16 = lax.broadcasted_iota(jnp.int32, (E, E), 1)
    ustrict = (r16 < c16).astype(jnp.float32)
    tile_base = lax.dot_general(tpe, ustrict, (((1,), (0,)), ((), ())),
                                preferred_element_type=jnp.float32)
    row_base = tile_base * TILE
    nt = jnp.sum(tpe)
    nt_ref[...] = jnp.broadcast_to(nt.astype(jnp.int32), (1, 1))

    trow = lax.broadcasted_iota(jnp.int32, (MAX_TILES, E), 0)
    t_eff = jnp.minimum(trow, nt.astype(jnp.int32) - 1)
    ge = (t_eff >= tile_base.astype(jnp.int32)).astype(jnp.int32)
    texp_ref[...] = jnp.sum(ge, axis=1, keepdims=True) - 1

    rr = lax.broadcasted_iota(jnp.int32, (RB, RB), 0)
    cc = lax.broadcasted_iota(jnp.int32, (RB, RB), 1)
    tri = (cc <= rr).astype(jnp.float32)

    def blk(b, base):
        ohb = oh_ref[pl.ds(b * RB, RB), :]
        csum = lax.dot_general(tri, ohb, (((1,), (0,)), ((), ())),
                               preferred_element_type=jnp.float32)
        rank = csum - 1.0 + base
        posb = jnp.sum((rank + row_base) * ohb, axis=1, keepdims=True)
        pos_ref[pl.ds(b * RB, RB), :] = posb.astype(jnp.int32)
        return base + csum[RB - 1:RB, :]

    lax.fori_loop(0, P // RB, blk, jnp.zeros((1, E), jnp.float32))


def _route(x_flat, w_router):
    return pl.pallas_call(
        _route_body,
        out_shape=[
            jax.ShapeDtypeStruct((P, 1), jnp.int32),
            jax.ShapeDtypeStruct((P, 128), jnp.float32),
            jax.ShapeDtypeStruct((MAX_TILES, 1), jnp.int32),
            jax.ShapeDtypeStruct((1, 1), jnp.int32),
        ],
        scratch_shapes=[pltpu.VMEM((P, E), jnp.float32)],
    )(x_flat, w_router)


def _dispatch_body(x_hbm, pos_hbm, dw_hbm, xs_hbm, ws_hbm,
                   pos0_v, pos1_v, rows_v, dw0_v, dw1_v, sem_in, sem_out):
    wid = lax.axis_index("s") * NC + lax.axis_index("c")
    n0 = wid * TPW
    c1 = pltpu.async_copy(pos_hbm.at[pl.ds(n0, TPW)], pos0_v, sem_in)
    c2 = pltpu.async_copy(pos_hbm.at[pl.ds(T + n0, TPW)], pos1_v, sem_in)
    c3 = pltpu.async_copy(dw_hbm.at[pl.ds(n0, TPW)], dw0_v, sem_in)
    c4 = pltpu.async_copy(dw_hbm.at[pl.ds(T + n0, TPW)], dw1_v, sem_in)
    c5 = pltpu.async_copy(x_hbm.at[pl.ds(n0, TPW)], rows_v, sem_in)
    c1.wait(); c2.wait(); c3.wait(); c4.wait(); c5.wait()
    s1 = pltpu.async_copy(rows_v, xs_hbm.at[pos0_v], sem_out)
    s2 = pltpu.async_copy(rows_v, xs_hbm.at[pos1_v], sem_out)
    s3 = pltpu.async_copy(dw0_v, ws_hbm.at[pos0_v], sem_out)
    s4 = pltpu.async_copy(dw1_v, ws_hbm.at[pos1_v], sem_out)
    s1.wait(); s2.wait(); s3.wait(); s4.wait()


@functools.cache
def _dispatch():
    return pl.kernel(
        _dispatch_body,
        out_type=[
            jax.ShapeDtypeStruct((MAX_ROWS, D), jnp.float32),
            jax.ShapeDtypeStruct((MAX_ROWS, 128), jnp.float32),
        ],
        mesh=plsc.VectorSubcoreMesh(core_axis_name="c", subcore_axis_name="s",
                                    num_cores=NC, num_subcores=NS),
        scratch_types=[
            pltpu.VMEM((TPW,), jnp.int32),
            pltpu.VMEM((TPW,), jnp.int32),
            pltpu.VMEM((TPW, D), jnp.float32),
            pltpu.VMEM((TPW, 128), jnp.float32),
            pltpu.VMEM((TPW, 128), jnp.float32),
            pltpu.SemaphoreType.DMA,
            pltpu.SemaphoreType.DMA,
        ],
    )


def _ffn_body(texp_s, nt_s, x_ref, ws_ref, wg_ref, wu_ref, wd_ref, o_ref):
    del texp_s
    t = pl.program_id(0)

    @pl.when(t < nt_s[0, 0])
    def _():
        x = x_ref[...].astype(jnp.bfloat16)
        g = lax.dot_general(x, wg_ref[0].astype(jnp.bfloat16),
                            (((1,), (1,)), ((), ())),
                            preferred_element_type=jnp.float32)
        u = lax.dot_general(x, wu_ref[0].astype(jnp.bfloat16),
                            (((1,), (1,)), ((), ())),
                            preferred_element_type=jnp.float32)
        h = g * (1.0 / (1.0 + jnp.exp(-g))) * u
        o = lax.dot_general(h.astype(jnp.bfloat16),
                            wd_ref[0].astype(jnp.bfloat16),
                            (((1,), (1,)), ((), ())),
                            preferred_element_type=jnp.float32)
        o_ref[...] = o * ws_ref[:, 0:1]


def _ffn(texp, nt, xs, ws, w_gate, w_up, w_down):
    grid_spec = pltpu.PrefetchScalarGridSpec(
        num_scalar_prefetch=2,
        grid=(MAX_TILES,),
        in_specs=[
            pl.BlockSpec((TILE, D),
                         lambda t, texp, nt: (jnp.minimum(t, nt[0, 0] - 1), 0)),
            pl.BlockSpec((TILE, 128),
                         lambda t, texp, nt: (jnp.minimum(t, nt[0, 0] - 1), 0)),
            pl.BlockSpec((1, F, D), lambda t, texp, nt: (texp[t, 0], 0, 0)),
            pl.BlockSpec((1, F, D), lambda t, texp, nt: (texp[t, 0], 0, 0)),
            pl.BlockSpec((1, D, F), lambda t, texp, nt: (texp[t, 0], 0, 0)),
        ],
        out_specs=pl.BlockSpec(
            (TILE, D), lambda t, texp, nt: (jnp.minimum(t, nt[0, 0] - 1), 0)),
    )
    return pl.pallas_call(
        _ffn_body,
        grid_spec=grid_spec,
        out_shape=jax.ShapeDtypeStruct((MAX_ROWS, D), jnp.float32),
    )(texp, nt, xs, ws, w_gate, w_up, w_down)


def _combine_body(os_hbm, pos_hbm, out_hbm,
                  pos_vs, ra_vs, rb_vs, semg0, semg1, semw):
    wid = lax.axis_index("s") * NC + lax.axis_index("c")
    semg = (semg0, semg1)
    gathers = []
    for c in range(TPW // CH):
        n0 = wid * TPW + c * CH
        pltpu.sync_copy(pos_hbm.at[pl.ds(n0, CH)], pos_vs[2 * c])
        pltpu.sync_copy(pos_hbm.at[pl.ds(T + n0, CH)], pos_vs[2 * c + 1])
        ga = pltpu.async_copy(os_hbm.at[pos_vs[2 * c]], ra_vs[c], semg[c])
        gb = pltpu.async_copy(os_hbm.at[pos_vs[2 * c + 1]], rb_vs[c], semg[c])
        gathers.append((ga, gb))

    writes = []
    for c in range(TPW // CH):
        n0 = wid * TPW + c * CH
        ga, gb = gathers[c]
        ga.wait(); gb.wait()
        ra_v, rb_v = ra_vs[c], rb_vs[c]

        def tokstep(j, carry):
            for q in range(D // 16):
                a = ra_v[j, pl.ds(q * 16, 16)]
                b = rb_v[j, pl.ds(q * 16, 16)]
                ra_v[j, pl.ds(q * 16, 16)] = a + b
            return carry

        lax.fori_loop(0, CH, tokstep, 0)
        writes.append(pltpu.async_copy(ra_v, out_hbm.at[pl.ds(n0, CH)], semw))
    for w in writes:
        w.wait()


@functools.cache
def _combine():
    nch = TPW // CH
    return pl.kernel(
        _combine_body,
        out_type=jax.ShapeDtypeStruct((T, D), jnp.float32),
        mesh=plsc.VectorSubcoreMesh(core_axis_name="c", subcore_axis_name="s",
                                    num_cores=NC, num_subcores=NS),
        scratch_types=[
            [pltpu.VMEM((CH,), jnp.int32) for _ in range(2 * nch)],
            [pltpu.VMEM((CH, D), jnp.float32) for _ in range(nch)],
            [pltpu.VMEM((CH, D), jnp.float32) for _ in range(nch)],
            pltpu.SemaphoreType.DMA,
            pltpu.SemaphoreType.DMA,
            pltpu.SemaphoreType.DMA,
        ],
    )


def kernel(x, W_router, W_gate, W_up, W_down):
    x_flat = x.reshape(T, D)
    pos, dw, texp, nt = _route(x_flat, W_router)
    pos_f = pos.reshape(P)
    xs, ws = _dispatch()(x_flat, pos_f, dw)
    os_ = _ffn(texp, nt, xs, ws, W_gate, W_up, W_down)
    out = _combine()(os_, pos_f)
    return out.reshape(x.shape)
